# baseline (speedup 1.0000x reference)
import functools

import jax
import jax.numpy as jnp
from jax import lax
from jax.experimental import pallas as pl
from jax.experimental.pallas import tpu as pltpu
from jax.experimental.pallas import tpu_sc as plsc

N = 10000
E = 320000
EMB = 128
G = 64
NCOORD3 = 273 * 3

NC = 2
NS = 16
NW = NC * NS
CH = 128
NCHUNK = E // CH
NP = 10240
ROWS_PER_TILE = NP // NS
HK = 80
NHCH = N // HK

NBUF = 4
NRB = 2
NXG = 3

_mesh = plsc.VectorSubcoreMesh(core_axis_name="c", subcore_axis_name="s")


def _make_edge_agg(fused_emb):
    out_type = [jax.ShapeDtypeStruct((NC, NP, EMB), jnp.float32)]
    scratch = (
        [pltpu.VMEM((2, CH), jnp.int32) for _ in range(NBUF)]
        + [pltpu.VMEM((CH, EMB), jnp.float32) for _ in range(NRB)]
        + [pltpu.VMEM_SHARED((NP, EMB), jnp.float32)]
        + [pltpu.SemaphoreType.DMA for _ in range(NBUF + 2 * NRB)]
    )
    if fused_emb:
        out_type.append(jax.ShapeDtypeStruct((N, EMB), jnp.float32))
        scratch += (
            [pltpu.VMEM((CH,), jnp.int32) for _ in range(NXG)]
            + [pltpu.SemaphoreType.DMA for _ in range(NXG)]
            + [pltpu.VMEM((HK,), jnp.int32),
               pltpu.VMEM((HK, EMB), jnp.float32),
               pltpu.SemaphoreType.DMA]
        )

    def body(ei_hbm, tab_hbm, zeros_hbm, *rest):
        if fused_emb:
            x_hbm, out_hbm, h0_hbm = rest[0], rest[1], rest[2]
            sc = rest[3:]
        else:
            out_hbm = rest[0]
            sc = rest[1:]
        idxb = sc[0:NBUF]
        rows = sc[NBUF:NBUF + NRB]
        agg_sh = sc[NBUF + NRB]
        p = NBUF + NRB + 1
        sem_i = sc[p:p + NBUF]
        sem_g = sc[p + NBUF:p + NBUF + NRB]
        sem_s = sc[p + NBUF + NRB:p + NBUF + 2 * NRB]
        if fused_emb:
            q = p + NBUF + 2 * NRB
            xg = sc[q:q + NXG]
            sem_x = sc[q + NXG:q + 2 * NXG]
            xg_h, hrows, sem_h = sc[q + 2 * NXG:q + 2 * NXG + 3]

        c = lax.axis_index("c")
        s = lax.axis_index("s")
        w = s * NC + c
        n_i = (NCHUNK - 1 - w) // NW + 1

        def off(i):
            return (w + NW * i) * CH

        def start_idx(i, b):
            pltpu.async_copy(ei_hbm.at[:, pl.ds(off(i), CH)], idxb[b],
                             sem_i[b])

        def wait_idx(i, b):
            pltpu.make_async_copy(ei_hbm.at[:, pl.ds(off(i), CH)], idxb[b],
                                  sem_i[b]).wait()

        def xg_desc(b, bx):
            return pltpu.make_async_copy(x_hbm.at[idxb[b].at[0]], xg[bx],
                                         sem_x[bx])

        def gather_desc(b, r, bx=None):
            src = xg[bx] if fused_emb else idxb[b].at[0]
            return pltpu.make_async_copy(tab_hbm.at[src], rows[r], sem_g[r])

        def scatter_desc(b, r):
            return pltpu.make_async_copy(rows[r], agg_sh.at[idxb[b].at[1]],
                                         sem_s[r])

        if fused_emb:
            start_idx(0, 0)
            start_idx(1, 1)
            start_idx(2, 2)
            wait_idx(0, 0)
            xg_desc(0, 0).start()
            wait_idx(1, 1)
            xg_desc(1, 1).start()
            xg_desc(0, 0).wait()
            gather_desc(0, 0, 0).start()
        else:
            start_idx(0, 0)
            start_idx(1, 1)
            wait_idx(0, 0)
            gather_desc(0, 0).start()

        pltpu.sync_copy(zeros_hbm,
                        agg_sh.at[pl.ds(s * ROWS_PER_TILE, ROWS_PER_TILE)])
        plsc.subcore_barrier()

        UNROLL = 12 if fused_emb else 4
        n_g = (n_i - 1) // UNROLL + 1

        def loop_body(g, carry):
            for j in range(UNROLL):
                i = g * UNROLL + j

                @pl.when(i < n_i)
                def _():
                    @pl.when(i >= 1)
                    def _():
                        scatter_desc((j - 1) % NBUF, (j - 1) % NRB).wait()

                    if fused_emb:
                        @pl.when(i + 3 < n_i)
                        def _():
                            start_idx(i + 3, (j + 3) % NBUF)

                        @pl.when(i + 2 < n_i)
                        def _():
                            wait_idx(i + 2, (j + 2) % NBUF)
                            xg_desc((j + 2) % NBUF, (j + 2) % NXG).start()

                        @pl.when(i + 1 < n_i)
                        def _():
                            xg_desc((j + 1) % NBUF, (j + 1) % NXG).wait()
                            gather_desc((j + 1) % NBUF, (j + 1) % NRB,
                                        (j + 1) % NXG).start()

                        gather_desc(j % NBUF, j % NRB, j % NXG).wait()
                    else:
                        @pl.when(i + 2 < n_i)
                        def _():
                            start_idx(i + 2, (j + 2) % NBUF)

                        @pl.when(i + 1 < n_i)
                        def _():
                            wait_idx(i + 1, (j + 1) % NBUF)
                            gather_desc((j + 1) % NBUF, (j + 1) % NRB).start()

                        gather_desc(j % NBUF, j % NRB).wait()

                    pltpu.async_copy(rows[j % NRB],
                                     agg_sh.at[idxb[j % NBUF].at[1]],
                                     sem_s[j % NRB], add=True)
            return carry

        lax.fori_loop(0, n_g, loop_body, 0)

        for b in range(NBUF):
            @pl.when(lax.rem(n_i - 1, NBUF) == b)
            def _():
                scatter_desc(b, b % NRB).wait()

        if fused_emb:
            for k in range(4):
                ch = w * 4 + k

                @pl.when(ch < NHCH)
                def _():
                    base = ch * HK
                    pltpu.sync_copy(x_hbm.at[pl.ds(base, HK)], xg_h)
                    pltpu.async_copy(tab_hbm.at[xg_h], hrows, sem_h).wait()
                    pltpu.sync_copy(hrows, h0_hbm.at[pl.ds(base, HK)])

        plsc.subcore_barrier()
        pltpu.sync_copy(
            agg_sh.at[pl.ds(s * ROWS_PER_TILE, ROWS_PER_TILE)],
            out_hbm.at[c, pl.ds(s * ROWS_PER_TILE, ROWS_PER_TILE)],
        )

    return pl.kernel(body, out_type=out_type if fused_emb else out_type[0],
                     mesh=_mesh, scratch_types=scratch)


_edge_agg = _make_edge_agg(fused_emb=False)
_edge_agg_emb = _make_edge_agg(fused_emb=True)


def _mlp_body(h_ref, a_ref, w1_ref, b1_ref, w2_ref, b2_ref, g_ref,
              bt_ref, o_ref, *, leaky):
    z = h_ref[...] + a_ref[0, 0:N, :] + a_ref[1, 0:N, :]
    z = jnp.dot(z, w1_ref[...], preferred_element_type=jnp.float32) + b1_ref[...]
    z = jnp.maximum(z, 0.0)
    z = jnp.dot(z, w2_ref[...], preferred_element_type=jnp.float32) + b2_ref[...]
    mu = jnp.mean(z, axis=0, keepdims=True)
    d = z - mu
    var = jnp.mean(d * d, axis=0, keepdims=True)
    zn = g_ref[...] * d * lax.rsqrt(var + 1e-5) + bt_ref[...]
    if leaky:
        zn = jnp.where(zn > 0, zn, 0.01 * zn)
    o_ref[...] = zn


def _final_body(h_ref, a_ref, w1_ref, b1_ref, w2_ref, b2_ref, g_ref,
                bt_ref, batch_ref, wc_ref, bc_ref, o_ref):
    z = h_ref[...] + a_ref[0, 0:N, :] + a_ref[1, 0:N, :]
    z = jnp.dot(z, w1_ref[...], preferred_element_type=jnp.float32) + b1_ref[...]
    z = jnp.maximum(z, 0.0)
    z = jnp.dot(z, w2_ref[...], preferred_element_type=jnp.float32) + b2_ref[...]
    mu = jnp.mean(z, axis=0, keepdims=True)
    d = z - mu
    var = jnp.mean(d * d, axis=0, keepdims=True)
    zn = g_ref[...] * d * lax.rsqrt(var + 1e-5) + bt_ref[...]

    onehot = jnp.where(
        batch_ref[...] == lax.broadcasted_iota(jnp.int32, (N, G), 1), 1.0, 0.0)
    cnt = jnp.maximum(jnp.sum(onehot, axis=0, keepdims=True), 1.0)
    oh_n = onehot / cnt
    rep = lax.dot_general(oh_n, zn, (((0,), (0,)), ((), ())),
                          preferred_element_type=jnp.float32)
    o_ref[...] = jnp.dot(rep, wc_ref[...],
                         preferred_element_type=jnp.float32) + bc_ref[...]


def _mlp_call(h, a, w1, b1, w2, b2, g, bt, leaky):
    return pl.pallas_call(
        functools.partial(_mlp_body, leaky=leaky),
        out_shape=jax.ShapeDtypeStruct((N, EMB), jnp.float32),
    )(h, a, w1, b1, w2, b2, g, bt)


def _final_call(h, a, w1, b1, w2, b2, g, bt, batch, wc, bc):
    return pl.pallas_call(
        _final_body,
        out_shape=jax.ShapeDtypeStruct((G, NCOORD3), jnp.float32),
    )(h, a, w1, b1, w2, b2, g, bt, batch, wc, bc)


def kernel(x, edge_index, batch, emb_table, W1_0, b1_0, W2_0, b2_0, gamma_0,
           beta_0, W1_1, b1_1, W2_1, b2_1, gamma_1, beta_1, W1_2, b1_2, W2_2,
           b2_2, gamma_2, beta_2, Wc, bc):
    zeros = jnp.zeros((ROWS_PER_TILE, EMB), jnp.float32)
    batch2 = batch.reshape(N, 1)

    params = [
        (W1_0, b1_0.reshape(1, EMB), W2_0, b2_0.reshape(1, EMB),
         gamma_0.reshape(1, EMB), beta_0.reshape(1, EMB)),
        (W1_1, b1_1.reshape(1, EMB), W2_1, b2_1.reshape(1, EMB),
         gamma_1.reshape(1, EMB), beta_1.reshape(1, EMB)),
        (W1_2, b1_2.reshape(1, EMB), W2_2, b2_2.reshape(1, EMB),
         gamma_2.reshape(1, EMB), beta_2.reshape(1, EMB)),
    ]

    agg, h = _edge_agg_emb(edge_index, emb_table, zeros, x.reshape(N))
    for l, (w1, b1, w2, b2, g, bt) in enumerate(params):
        if l > 0:
            agg = _edge_agg(edge_index, h, zeros)
        if l < 2:
            h = _mlp_call(h, agg, w1, b1, w2, b2, g, bt, leaky=True)
        else:
            coords = _final_call(h, agg, w1, b1, w2, b2, g, bt,
                                 batch2, Wc, bc.reshape(1, NCOORD3))
    return coords.reshape(-1, 3)


# speedup vs baseline: 1.0801x; 1.0801x over previous
import functools

import jax
import jax.numpy as jnp
from jax import lax
from jax.experimental import pallas as pl
from jax.experimental.pallas import tpu as pltpu
from jax.experimental.pallas import tpu_sc as plsc

N = 10000
E = 320000
EMB = 128
G = 64
NCOORD3 = 273 * 3

NC = 2
NS = 16
NW = NC * NS
CH = 128
NCHUNK = E // CH
NP = 10240
ROWS_PER_TILE = NP // NS
HK = 80
NHCH = N // HK

NBUF = 4
NRB = 2
NXG = 3

_mesh = plsc.VectorSubcoreMesh(core_axis_name="c", subcore_axis_name="s")


def _make_edge_agg(fused_emb):
    out_type = [jax.ShapeDtypeStruct((NC, NP, EMB), jnp.float32)]
    scratch = (
        [pltpu.VMEM((2, CH), jnp.int32) for _ in range(NBUF)]
        + [pltpu.VMEM((CH, EMB), jnp.float32) for _ in range(NRB)]
        + [pltpu.VMEM_SHARED((NP, EMB), jnp.float32)]
        + [pltpu.SemaphoreType.DMA for _ in range(NBUF + 2 * NRB)]
    )
    if fused_emb:
        out_type.append(jax.ShapeDtypeStruct((N, EMB), jnp.float32))
        scratch += (
            [pltpu.VMEM((CH,), jnp.int32) for _ in range(NXG)]
            + [pltpu.SemaphoreType.DMA for _ in range(NXG)]
            + [pltpu.VMEM((HK,), jnp.int32),
               pltpu.VMEM((HK, EMB), jnp.float32),
               pltpu.SemaphoreType.DMA]
        )

    def body(ei_hbm, tab_hbm, zeros_hbm, *rest):
        if fused_emb:
            x_hbm, out_hbm, h0_hbm = rest[0], rest[1], rest[2]
            sc = rest[3:]
        else:
            out_hbm = rest[0]
            sc = rest[1:]
        idxb = sc[0:NBUF]
        rows = sc[NBUF:NBUF + NRB]
        agg_sh = sc[NBUF + NRB]
        p = NBUF + NRB + 1
        sem_i = sc[p:p + NBUF]
        sem_g = sc[p + NBUF:p + NBUF + NRB]
        sem_s = sc[p + NBUF + NRB:p + NBUF + 2 * NRB]
        if fused_emb:
            q = p + NBUF + 2 * NRB
            xg = sc[q:q + NXG]
            sem_x = sc[q + NXG:q + 2 * NXG]
            xg_h, hrows, sem_h = sc[q + 2 * NXG:q + 2 * NXG + 3]

        c = lax.axis_index("c")
        s = lax.axis_index("s")
        w = s * NC + c
        n_i = (NCHUNK - 1 - w) // NW + 1

        def off(i):
            return (w + NW * i) * CH

        def start_idx(i, b):
            pltpu.async_copy(ei_hbm.at[:, pl.ds(off(i), CH)], idxb[b],
                             sem_i[b])

        def wait_idx(i, b):
            pltpu.make_async_copy(ei_hbm.at[:, pl.ds(off(i), CH)], idxb[b],
                                  sem_i[b]).wait()

        def xg_desc(b, bx):
            return pltpu.make_async_copy(x_hbm.at[idxb[b].at[0]], xg[bx],
                                         sem_x[bx])

        def gather_desc(b, r, bx=None):
            src = xg[bx] if fused_emb else idxb[b].at[0]
            return pltpu.make_async_copy(tab_hbm.at[src], rows[r], sem_g[r])

        def scatter_desc(b, r):
            return pltpu.make_async_copy(rows[r], agg_sh.at[idxb[b].at[1]],
                                         sem_s[r])

        if fused_emb:
            start_idx(0, 0)
            start_idx(1, 1)
            start_idx(2, 2)
            wait_idx(0, 0)
            xg_desc(0, 0).start()
            wait_idx(1, 1)
            xg_desc(1, 1).start()
            xg_desc(0, 0).wait()
            gather_desc(0, 0, 0).start()
        else:
            start_idx(0, 0)
            start_idx(1, 1)
            wait_idx(0, 0)
            gather_desc(0, 0).start()

        pltpu.sync_copy(zeros_hbm,
                        agg_sh.at[pl.ds(s * ROWS_PER_TILE, ROWS_PER_TILE)])
        plsc.subcore_barrier()

        UNROLL = 12 if fused_emb else 4
        n_g = (n_i - 1) // UNROLL + 1

        def loop_body(g, carry):
            for j in range(UNROLL):
                i = g * UNROLL + j

                @pl.when(i < n_i)
                def _():
                    @pl.when(i >= 1)
                    def _():
                        scatter_desc((j - 1) % NBUF, (j - 1) % NRB).wait()

                    if fused_emb:
                        @pl.when(i + 3 < n_i)
                        def _():
                            start_idx(i + 3, (j + 3) % NBUF)

                        @pl.when(i + 2 < n_i)
                        def _():
                            wait_idx(i + 2, (j + 2) % NBUF)
                            xg_desc((j + 2) % NBUF, (j + 2) % NXG).start()

                        @pl.when(i + 1 < n_i)
                        def _():
                            xg_desc((j + 1) % NBUF, (j + 1) % NXG).wait()
                            gather_desc((j + 1) % NBUF, (j + 1) % NRB,
                                        (j + 1) % NXG).start()

                        gather_desc(j % NBUF, j % NRB, j % NXG).wait()
                    else:
                        @pl.when(i + 2 < n_i)
                        def _():
                            start_idx(i + 2, (j + 2) % NBUF)

                        @pl.when(i + 1 < n_i)
                        def _():
                            wait_idx(i + 1, (j + 1) % NBUF)
                            gather_desc((j + 1) % NBUF, (j + 1) % NRB).start()

                        gather_desc(j % NBUF, j % NRB).wait()

                    pltpu.async_copy(rows[j % NRB],
                                     agg_sh.at[idxb[j % NBUF].at[1]],
                                     sem_s[j % NRB], add=True)
            return carry

        lax.fori_loop(0, n_g, loop_body, 0)

        for b in range(NBUF):
            @pl.when(lax.rem(n_i - 1, NBUF) == b)
            def _():
                scatter_desc(b, b % NRB).wait()

        if fused_emb:
            for k in range(4):
                ch = w * 4 + k

                @pl.when(ch < NHCH)
                def _():
                    base = ch * HK
                    pltpu.sync_copy(x_hbm.at[pl.ds(base, HK)], xg_h)
                    pltpu.async_copy(tab_hbm.at[xg_h], hrows, sem_h).wait()
                    pltpu.sync_copy(hrows, h0_hbm.at[pl.ds(base, HK)])

        plsc.subcore_barrier()
        pltpu.sync_copy(
            agg_sh.at[pl.ds(s * ROWS_PER_TILE, ROWS_PER_TILE)],
            out_hbm.at[c, pl.ds(s * ROWS_PER_TILE, ROWS_PER_TILE)],
        )

    return pl.kernel(body, out_type=out_type if fused_emb else out_type[0],
                     mesh=_mesh, scratch_types=scratch)


_edge_agg = _make_edge_agg(fused_emb=False)


@functools.partial(
    pl.kernel,
    out_type=jax.ShapeDtypeStruct((N, EMB), jnp.float32),
    mesh=_mesh,
    scratch_types=[
        pltpu.VMEM((HK,), jnp.int32),
        pltpu.VMEM((HK, EMB), jnp.float32),
        pltpu.SemaphoreType.DMA,
    ],
)
def _emb_gather(x_hbm, emb_hbm, out_hbm, xg_v, rows_v, sem):
    c = lax.axis_index("c")
    s = lax.axis_index("s")
    w = s * NC + c
    for k in range(4):
        ch = w * 4 + k

        @pl.when(ch < NHCH)
        def _():
            base = ch * HK
            pltpu.sync_copy(x_hbm.at[pl.ds(base, HK)], xg_v)
            pltpu.async_copy(emb_hbm.at[xg_v], rows_v, sem).wait()
            pltpu.sync_copy(rows_v, out_hbm.at[pl.ds(base, HK)])


def _mlp_body(h_ref, a_ref, w1_ref, b1_ref, w2_ref, b2_ref, g_ref,
              bt_ref, o_ref, *, leaky):
    z = h_ref[...] + a_ref[0, 0:N, :] + a_ref[1, 0:N, :]
    z = jnp.dot(z, w1_ref[...], preferred_element_type=jnp.float32) + b1_ref[...]
    z = jnp.maximum(z, 0.0)
    z = jnp.dot(z, w2_ref[...], preferred_element_type=jnp.float32) + b2_ref[...]
    mu = jnp.mean(z, axis=0, keepdims=True)
    d = z - mu
    var = jnp.mean(d * d, axis=0, keepdims=True)
    zn = g_ref[...] * d * lax.rsqrt(var + 1e-5) + bt_ref[...]
    if leaky:
        zn = jnp.where(zn > 0, zn, 0.01 * zn)
    o_ref[...] = zn


def _final_body(h_ref, a_ref, w1_ref, b1_ref, w2_ref, b2_ref, g_ref,
                bt_ref, batch_ref, wc_ref, bc_ref, o_ref):
    z = h_ref[...] + a_ref[0, 0:N, :] + a_ref[1, 0:N, :]
    z = jnp.dot(z, w1_ref[...], preferred_element_type=jnp.float32) + b1_ref[...]
    z = jnp.maximum(z, 0.0)
    z = jnp.dot(z, w2_ref[...], preferred_element_type=jnp.float32) + b2_ref[...]
    mu = jnp.mean(z, axis=0, keepdims=True)
    d = z - mu
    var = jnp.mean(d * d, axis=0, keepdims=True)
    zn = g_ref[...] * d * lax.rsqrt(var + 1e-5) + bt_ref[...]

    onehot = jnp.where(
        batch_ref[...] == lax.broadcasted_iota(jnp.int32, (N, G), 1), 1.0, 0.0)
    cnt = jnp.maximum(jnp.sum(onehot, axis=0, keepdims=True), 1.0)
    oh_n = onehot / cnt
    rep = lax.dot_general(oh_n, zn, (((0,), (0,)), ((), ())),
                          preferred_element_type=jnp.float32)
    o_ref[...] = jnp.dot(rep, wc_ref[...],
                         preferred_element_type=jnp.float32) + bc_ref[...]


def _mlp_call(h, a, w1, b1, w2, b2, g, bt, leaky):
    return pl.pallas_call(
        functools.partial(_mlp_body, leaky=leaky),
        out_shape=jax.ShapeDtypeStruct((N, EMB), jnp.float32),
    )(h, a, w1, b1, w2, b2, g, bt)


def _final_call(h, a, w1, b1, w2, b2, g, bt, batch, wc, bc):
    return pl.pallas_call(
        _final_body,
        out_shape=jax.ShapeDtypeStruct((G, NCOORD3), jnp.float32),
    )(h, a, w1, b1, w2, b2, g, bt, batch, wc, bc)


def kernel(x, edge_index, batch, emb_table, W1_0, b1_0, W2_0, b2_0, gamma_0,
           beta_0, W1_1, b1_1, W2_1, b2_1, gamma_1, beta_1, W1_2, b1_2, W2_2,
           b2_2, gamma_2, beta_2, Wc, bc):
    zeros = jnp.zeros((ROWS_PER_TILE, EMB), jnp.float32)
    batch2 = batch.reshape(N, 1)

    params = [
        (W1_0, b1_0.reshape(1, EMB), W2_0, b2_0.reshape(1, EMB),
         gamma_0.reshape(1, EMB), beta_0.reshape(1, EMB)),
        (W1_1, b1_1.reshape(1, EMB), W2_1, b2_1.reshape(1, EMB),
         gamma_1.reshape(1, EMB), beta_1.reshape(1, EMB)),
        (W1_2, b1_2.reshape(1, EMB), W2_2, b2_2.reshape(1, EMB),
         gamma_2.reshape(1, EMB), beta_2.reshape(1, EMB)),
    ]

    h = _emb_gather(x.reshape(N), emb_table)
    for l, (w1, b1, w2, b2, g, bt) in enumerate(params):
        agg = _edge_agg(edge_index, h, zeros)
        if l < 2:
            h = _mlp_call(h, agg, w1, b1, w2, b2, g, bt, leaky=True)
        else:
            coords = _final_call(h, agg, w1, b1, w2, b2, g, bt,
                                 batch2, Wc, bc.reshape(1, NCOORD3))
    return coords.reshape(-1, 3)


# speedup vs baseline: 1.1943x; 1.1057x over previous
import functools

import jax
import jax.numpy as jnp
from jax import lax
from jax.experimental import pallas as pl
from jax.experimental.pallas import tpu as pltpu
from jax.experimental.pallas import tpu_sc as plsc

N = 10000
E = 320000
EMB = 128
G = 64
NCOORD3 = 273 * 3

NC = 2
NS = 16
NW = NC * NS
CH = 128
NCHUNK = E // CH
NP = 10240
ROWS_PER_TILE = NP // NS
HK = 80
NHCH = N // HK

NBUF = 4
NRB = 3
DUMP = 640
NXG = 3

_mesh = plsc.VectorSubcoreMesh(core_axis_name="c", subcore_axis_name="s")


def _make_edge_agg(fused_emb):
    out_type = [jax.ShapeDtypeStruct((NC, N, EMB), jnp.float32)]
    scratch = (
        [pltpu.VMEM((2, CH), jnp.int32) for _ in range(NBUF)]
        + [pltpu.VMEM((CH, EMB), jnp.float32) for _ in range(NRB)]
        + [pltpu.VMEM_SHARED((N, EMB), jnp.float32)]
        + [pltpu.SemaphoreType.DMA for _ in range(NBUF + 2 * NRB)]
    )
    if fused_emb:
        out_type.append(jax.ShapeDtypeStruct((N, EMB), jnp.float32))
        scratch += (
            [pltpu.VMEM((CH,), jnp.int32) for _ in range(NXG)]
            + [pltpu.SemaphoreType.DMA for _ in range(NXG)]
            + [pltpu.VMEM((HK,), jnp.int32),
               pltpu.VMEM((HK, EMB), jnp.float32),
               pltpu.SemaphoreType.DMA]
        )

    def body(ei_hbm, tab_hbm, zeros_hbm, *rest):
        if fused_emb:
            x_hbm, out_hbm, h0_hbm = rest[0], rest[1], rest[2]
            sc = rest[3:]
        else:
            out_hbm = rest[0]
            sc = rest[1:]
        idxb = sc[0:NBUF]
        rows = sc[NBUF:NBUF + NRB]
        agg_sh = sc[NBUF + NRB]
        p = NBUF + NRB + 1
        sem_i = sc[p:p + NBUF]
        sem_g = sc[p + NBUF:p + NBUF + NRB]
        sem_s = sc[p + NBUF + NRB:p + NBUF + 2 * NRB]
        if fused_emb:
            q = p + NBUF + 2 * NRB
            xg = sc[q:q + NXG]
            sem_x = sc[q + NXG:q + 2 * NXG]
            xg_h, hrows, sem_h = sc[q + 2 * NXG:q + 2 * NXG + 3]

        c = lax.axis_index("c")
        s = lax.axis_index("s")
        w = s * NC + c
        n_i = (NCHUNK - 1 - w) // NW + 1

        def off(i):
            return (w + NW * i) * CH

        def start_idx(i, b):
            pltpu.async_copy(ei_hbm.at[:, pl.ds(off(i), CH)], idxb[b],
                             sem_i[b])

        def wait_idx(i, b):
            pltpu.make_async_copy(ei_hbm.at[:, pl.ds(off(i), CH)], idxb[b],
                                  sem_i[b]).wait()

        def xg_desc(b, bx):
            return pltpu.make_async_copy(x_hbm.at[idxb[b].at[0]], xg[bx],
                                         sem_x[bx])

        def gather_desc(b, r, bx=None):
            src = xg[bx] if fused_emb else idxb[b].at[0]
            return pltpu.make_async_copy(tab_hbm.at[src], rows[r], sem_g[r])

        def scatter_desc(b, r):
            return pltpu.make_async_copy(rows[r], agg_sh.at[idxb[b].at[1]],
                                         sem_s[r])

        if fused_emb:
            start_idx(0, 0)
            start_idx(1, 1)
            start_idx(2, 2)
            wait_idx(0, 0)
            xg_desc(0, 0).start()
            wait_idx(1, 1)
            xg_desc(1, 1).start()
            xg_desc(0, 0).wait()
            gather_desc(0, 0, 0).start()
        else:
            start_idx(0, 0)
            start_idx(1, 1)
            start_idx(2, 2)
            wait_idx(0, 0)
            gather_desc(0, 0).start()
            wait_idx(1, 1)
            gather_desc(1, 1).start()

        @pl.when(s < NS - 1)
        def _():
            pltpu.sync_copy(zeros_hbm, agg_sh.at[pl.ds(s * DUMP, DUMP)])

        @pl.when(s == NS - 1)
        def _():
            pltpu.sync_copy(zeros_hbm.at[pl.ds(0, N - (NS - 1) * DUMP)],
                            agg_sh.at[pl.ds((NS - 1) * DUMP,
                                            N - (NS - 1) * DUMP)])

        plsc.subcore_barrier()

        UNROLL = 12
        n_g = (n_i - 1) // UNROLL + 1

        def loop_body(g, carry):
            for j in range(UNROLL):
                i = g * UNROLL + j

                @pl.when(i < n_i)
                def _():
                    @pl.when(i >= 1)
                    def _():
                        scatter_desc((j - 1) % NBUF, (j - 1) % NRB).wait()

                    if fused_emb:
                        @pl.when(i + 3 < n_i)
                        def _():
                            start_idx(i + 3, (j + 3) % NBUF)

                        @pl.when(i + 2 < n_i)
                        def _():
                            wait_idx(i + 2, (j + 2) % NBUF)
                            xg_desc((j + 2) % NBUF, (j + 2) % NXG).start()

                        @pl.when(i + 1 < n_i)
                        def _():
                            xg_desc((j + 1) % NBUF, (j + 1) % NXG).wait()
                            gather_desc((j + 1) % NBUF, (j + 1) % NRB,
                                        (j + 1) % NXG).start()

                        gather_desc(j % NBUF, j % NRB, j % NXG).wait()
                    else:
                        @pl.when(i + 3 < n_i)
                        def _():
                            start_idx(i + 3, (j + 3) % NBUF)

                        @pl.when(i + 2 < n_i)
                        def _():
                            wait_idx(i + 2, (j + 2) % NBUF)
                            gather_desc((j + 2) % NBUF, (j + 2) % NRB).start()

                        gather_desc(j % NBUF, j % NRB).wait()

                    pltpu.async_copy(rows[j % NRB],
                                     agg_sh.at[idxb[j % NBUF].at[1]],
                                     sem_s[j % NRB], add=True)
            return carry

        lax.fori_loop(0, n_g, loop_body, 0)

        for b in range(NBUF):
            for r in range(NRB):
                @pl.when((lax.rem(n_i - 1, NBUF) == b)
                         & (lax.rem(n_i - 1, NRB) == r))
                def _():
                    scatter_desc(b, r).wait()

        if fused_emb:
            for k in range(4):
                ch = w * 4 + k

                @pl.when(ch < NHCH)
                def _():
                    base = ch * HK
                    pltpu.sync_copy(x_hbm.at[pl.ds(base, HK)], xg_h)
                    pltpu.async_copy(tab_hbm.at[xg_h], hrows, sem_h).wait()
                    pltpu.sync_copy(hrows, h0_hbm.at[pl.ds(base, HK)])

        plsc.subcore_barrier()

        @pl.when(s < NS - 1)
        def _():
            pltpu.sync_copy(agg_sh.at[pl.ds(s * DUMP, DUMP)],
                            out_hbm.at[c, pl.ds(s * DUMP, DUMP)])

        @pl.when(s == NS - 1)
        def _():
            tail = N - (NS - 1) * DUMP
            pltpu.sync_copy(agg_sh.at[pl.ds((NS - 1) * DUMP, tail)],
                            out_hbm.at[c, pl.ds((NS - 1) * DUMP, tail)])

    return pl.kernel(body, out_type=out_type if fused_emb else out_type[0],
                     mesh=_mesh, scratch_types=scratch)


_edge_agg = _make_edge_agg(fused_emb=False)


@functools.partial(
    pl.kernel,
    out_type=jax.ShapeDtypeStruct((N, EMB), jnp.float32),
    mesh=_mesh,
    scratch_types=[
        pltpu.VMEM((HK,), jnp.int32),
        pltpu.VMEM((HK, EMB), jnp.float32),
        pltpu.SemaphoreType.DMA,
    ],
)
def _emb_gather(x_hbm, emb_hbm, out_hbm, xg_v, rows_v, sem):
    c = lax.axis_index("c")
    s = lax.axis_index("s")
    w = s * NC + c
    for k in range(4):
        ch = w * 4 + k

        @pl.when(ch < NHCH)
        def _():
            base = ch * HK
            pltpu.sync_copy(x_hbm.at[pl.ds(base, HK)], xg_v)
            pltpu.async_copy(emb_hbm.at[xg_v], rows_v, sem).wait()
            pltpu.sync_copy(rows_v, out_hbm.at[pl.ds(base, HK)])


def _mlp_body(h_ref, a_ref, w1_ref, b1_ref, w2_ref, b2_ref, g_ref,
              bt_ref, o_ref, *, leaky):
    z = h_ref[...] + a_ref[0, 0:N, :] + a_ref[1, 0:N, :]
    z = jnp.dot(z, w1_ref[...], preferred_element_type=jnp.float32) + b1_ref[...]
    z = jnp.maximum(z, 0.0)
    z = jnp.dot(z, w2_ref[...], preferred_element_type=jnp.float32) + b2_ref[...]
    mu = jnp.mean(z, axis=0, keepdims=True)
    d = z - mu
    var = jnp.mean(d * d, axis=0, keepdims=True)
    zn = g_ref[...] * d * lax.rsqrt(var + 1e-5) + bt_ref[...]
    if leaky:
        zn = jnp.where(zn > 0, zn, 0.01 * zn)
    o_ref[...] = zn


def _final_body(h_ref, a_ref, w1_ref, b1_ref, w2_ref, b2_ref, g_ref,
                bt_ref, batch_ref, wc_ref, bc_ref, o_ref):
    z = h_ref[...] + a_ref[0, 0:N, :] + a_ref[1, 0:N, :]
    z = jnp.dot(z, w1_ref[...], preferred_element_type=jnp.float32) + b1_ref[...]
    z = jnp.maximum(z, 0.0)
    z = jnp.dot(z, w2_ref[...], preferred_element_type=jnp.float32) + b2_ref[...]
    mu = jnp.mean(z, axis=0, keepdims=True)
    d = z - mu
    var = jnp.mean(d * d, axis=0, keepdims=True)
    zn = g_ref[...] * d * lax.rsqrt(var + 1e-5) + bt_ref[...]

    onehot = jnp.where(
        batch_ref[...] == lax.broadcasted_iota(jnp.int32, (N, G), 1), 1.0, 0.0)
    cnt = jnp.maximum(jnp.sum(onehot, axis=0, keepdims=True), 1.0)
    oh_n = onehot / cnt
    rep = lax.dot_general(oh_n, zn, (((0,), (0,)), ((), ())),
                          preferred_element_type=jnp.float32)
    o_ref[...] = jnp.dot(rep, wc_ref[...],
                         preferred_element_type=jnp.float32) + bc_ref[...]


def _mlp_call(h, a, w1, b1, w2, b2, g, bt, leaky):
    return pl.pallas_call(
        functools.partial(_mlp_body, leaky=leaky),
        out_shape=jax.ShapeDtypeStruct((N, EMB), jnp.float32),
    )(h, a, w1, b1, w2, b2, g, bt)


def _final_call(h, a, w1, b1, w2, b2, g, bt, batch, wc, bc):
    return pl.pallas_call(
        _final_body,
        out_shape=jax.ShapeDtypeStruct((G, NCOORD3), jnp.float32),
    )(h, a, w1, b1, w2, b2, g, bt, batch, wc, bc)


def kernel(x, edge_index, batch, emb_table, W1_0, b1_0, W2_0, b2_0, gamma_0,
           beta_0, W1_1, b1_1, W2_1, b2_1, gamma_1, beta_1, W1_2, b1_2, W2_2,
           b2_2, gamma_2, beta_2, Wc, bc):
    zeros = jnp.zeros((ROWS_PER_TILE, EMB), jnp.float32)
    batch2 = batch.reshape(N, 1)

    params = [
        (W1_0, b1_0.reshape(1, EMB), W2_0, b2_0.reshape(1, EMB),
         gamma_0.reshape(1, EMB), beta_0.reshape(1, EMB)),
        (W1_1, b1_1.reshape(1, EMB), W2_1, b2_1.reshape(1, EMB),
         gamma_1.reshape(1, EMB), beta_1.reshape(1, EMB)),
        (W1_2, b1_2.reshape(1, EMB), W2_2, b2_2.reshape(1, EMB),
         gamma_2.reshape(1, EMB), beta_2.reshape(1, EMB)),
    ]

    h = _emb_gather(x.reshape(N), emb_table)
    for l, (w1, b1, w2, b2, g, bt) in enumerate(params):
        agg = _edge_agg(edge_index, h, zeros)
        if l < 2:
            h = _mlp_call(h, agg, w1, b1, w2, b2, g, bt, leaky=True)
        else:
            coords = _final_call(h, agg, w1, b1, w2, b2, g, bt,
                                 batch2, Wc, bc.reshape(1, NCOORD3))
    return coords.reshape(-1, 3)


# speedup vs baseline: 1.1975x; 1.0027x over previous
import functools

import jax
import jax.numpy as jnp
from jax import lax
from jax.experimental import pallas as pl
from jax.experimental.pallas import tpu as pltpu
from jax.experimental.pallas import tpu_sc as plsc

N = 10000
E = 320000
EMB = 128
G = 64
NCOORD3 = 273 * 3

NC = 2
NS = 16
NW = NC * NS
CH = 128
NCHUNK = E // CH
DUMP = 640
TAIL = N - (NS - 1) * DUMP
HK = 80
NHCH = N // HK

NBUF = 4
NRB = 3

_mesh = plsc.VectorSubcoreMesh(core_axis_name="c", subcore_axis_name="s")


@functools.partial(
    pl.kernel,
    out_type=jax.ShapeDtypeStruct((N, EMB), jnp.float32),
    mesh=_mesh,
    scratch_types=(
        [pltpu.VMEM((HK,), jnp.int32) for _ in range(2)]
        + [pltpu.VMEM((HK, EMB), jnp.float32) for _ in range(2)]
        + [pltpu.SemaphoreType.DMA for _ in range(4)]
    ),
)
def _emb_gather(x_hbm, emb_hbm, out_hbm, xg0, xg1, r0, r1, sg0, sg1, sw0, sw1):
    c = lax.axis_index("c")
    s = lax.axis_index("s")
    w = s * NC + c
    xg, rows = [xg0, xg1], [r0, r1]
    sg, sw = [sg0, sg1], [sw0, sw1]

    for k in range(4):
        ch = w * 4 + k

        @pl.when(ch < NHCH)
        def _():
            base = ch * HK
            pltpu.sync_copy(x_hbm.at[pl.ds(base, HK)], xg[k % 2])
            pltpu.async_copy(emb_hbm.at[xg[k % 2]], rows[k % 2],
                             sg[k % 2]).wait()
            pltpu.sync_copy(rows[k % 2], out_hbm.at[pl.ds(base, HK)])


@functools.partial(
    pl.kernel,
    out_type=jax.ShapeDtypeStruct((NC, N, EMB), jnp.float32),
    mesh=_mesh,
    scratch_types=(
        [pltpu.VMEM((2, CH), jnp.int32) for _ in range(NBUF)]
        + [pltpu.VMEM((CH, EMB), jnp.float32) for _ in range(NRB)]
        + [pltpu.VMEM_SHARED((N, EMB), jnp.float32)]
        + [pltpu.SemaphoreType.DMA for _ in range(NBUF + 2 * NRB)]
    ),
)
def _edge_agg(ei_hbm, h_hbm, zeros_hbm, out_hbm, *sc):
    idxb = sc[0:NBUF]
    rows = sc[NBUF:NBUF + NRB]
    agg_sh = sc[NBUF + NRB]
    p = NBUF + NRB + 1
    sem_i = sc[p:p + NBUF]
    sem_g = sc[p + NBUF:p + NBUF + NRB]
    sem_s = sc[p + NBUF + NRB:p + NBUF + 2 * NRB]

    c = lax.axis_index("c")
    s = lax.axis_index("s")
    w = s * NC + c
    n_i = (NCHUNK - 1 - w) // NW + 1

    def off(i):
        return (w + NW * i) * CH

    def start_idx(i, b):
        pltpu.async_copy(ei_hbm.at[:, pl.ds(off(i), CH)], idxb[b], sem_i[b])

    def wait_idx(i, b):
        pltpu.make_async_copy(ei_hbm.at[:, pl.ds(off(i), CH)], idxb[b],
                              sem_i[b]).wait()

    def gather_desc(b, r):
        return pltpu.make_async_copy(h_hbm.at[idxb[b].at[0]], rows[r],
                                     sem_g[r])

    def scatter_desc(b, r):
        return pltpu.make_async_copy(rows[r], agg_sh.at[idxb[b].at[1]],
                                     sem_s[r])

    start_idx(0, 0)
    start_idx(1, 1)
    start_idx(2, 2)
    wait_idx(0, 0)
    gather_desc(0, 0).start()
    wait_idx(1, 1)
    gather_desc(1, 1).start()

    @pl.when(s < NS - 1)
    def _():
        pltpu.sync_copy(zeros_hbm, agg_sh.at[pl.ds(s * DUMP, DUMP)])

    @pl.when(s == NS - 1)
    def _():
        pltpu.sync_copy(zeros_hbm.at[pl.ds(0, TAIL)],
                        agg_sh.at[pl.ds((NS - 1) * DUMP, TAIL)])

    plsc.subcore_barrier()

    UNROLL = 12
    n_g = (n_i - 1) // UNROLL + 1

    def loop_body(g, carry):
        for j in range(UNROLL):
            i = g * UNROLL + j

            @pl.when(i < n_i)
            def _():
                @pl.when(i >= 1)
                def _():
                    scatter_desc((j - 1) % NBUF, (j - 1) % NRB).wait()

                @pl.when(i + 3 < n_i)
                def _():
                    start_idx(i + 3, (j + 3) % NBUF)

                @pl.when(i + 2 < n_i)
                def _():
                    wait_idx(i + 2, (j + 2) % NBUF)
                    gather_desc((j + 2) % NBUF, (j + 2) % NRB).start()

                gather_desc(j % NBUF, j % NRB).wait()
                pltpu.async_copy(rows[j % NRB], agg_sh.at[idxb[j % NBUF].at[1]],
                                 sem_s[j % NRB], add=True)
        return carry

    lax.fori_loop(0, n_g, loop_body, 0)

    for b in range(NBUF):
        for r in range(NRB):
            @pl.when((lax.rem(n_i - 1, NBUF) == b)
                     & (lax.rem(n_i - 1, NRB) == r))
            def _():
                scatter_desc(b, r).wait()

    plsc.subcore_barrier()

    @pl.when(s < NS - 1)
    def _():
        pltpu.sync_copy(agg_sh.at[pl.ds(s * DUMP, DUMP)],
                        out_hbm.at[c, pl.ds(s * DUMP, DUMP)])

    @pl.when(s == NS - 1)
    def _():
        pltpu.sync_copy(agg_sh.at[pl.ds((NS - 1) * DUMP, TAIL)],
                        out_hbm.at[c, pl.ds((NS - 1) * DUMP, TAIL)])


def _mlp_body(h_ref, a_ref, w1_ref, b1_ref, w2_ref, b2_ref, g_ref,
              bt_ref, o_ref, *, leaky):
    z = h_ref[...] + a_ref[0] + a_ref[1]
    z = jnp.dot(z, w1_ref[...], preferred_element_type=jnp.float32) + b1_ref[...]
    z = jnp.maximum(z, 0.0)
    z = jnp.dot(z, w2_ref[...], preferred_element_type=jnp.float32) + b2_ref[...]
    mu = jnp.mean(z, axis=0, keepdims=True)
    d = z - mu
    var = jnp.mean(d * d, axis=0, keepdims=True)
    zn = g_ref[...] * d * lax.rsqrt(var + 1e-5) + bt_ref[...]
    if leaky:
        zn = jnp.where(zn > 0, zn, 0.01 * zn)
    o_ref[...] = zn


def _final_body(h_ref, a_ref, w1_ref, b1_ref, w2_ref, b2_ref, g_ref,
                bt_ref, batch_ref, wc_ref, bc_ref, o_ref):
    z = h_ref[...] + a_ref[0] + a_ref[1]
    z = jnp.dot(z, w1_ref[...], preferred_element_type=jnp.float32) + b1_ref[...]
    z = jnp.maximum(z, 0.0)
    z = jnp.dot(z, w2_ref[...], preferred_element_type=jnp.float32) + b2_ref[...]
    mu = jnp.mean(z, axis=0, keepdims=True)
    d = z - mu
    var = jnp.mean(d * d, axis=0, keepdims=True)
    zn = g_ref[...] * d * lax.rsqrt(var + 1e-5) + bt_ref[...]

    onehot = jnp.where(
        batch_ref[...] == lax.broadcasted_iota(jnp.int32, (N, G), 1), 1.0, 0.0)
    cnt = jnp.maximum(jnp.sum(onehot, axis=0, keepdims=True), 1.0)
    oh_n = onehot / cnt
    rep = lax.dot_general(oh_n, zn, (((0,), (0,)), ((), ())),
                          preferred_element_type=jnp.float32)
    o_ref[...] = jnp.dot(rep, wc_ref[...],
                         preferred_element_type=jnp.float32) + bc_ref[...]


def _mlp_call(h, a, w1, b1, w2, b2, g, bt, leaky):
    return pl.pallas_call(
        functools.partial(_mlp_body, leaky=leaky),
        out_shape=jax.ShapeDtypeStruct((N, EMB), jnp.float32),
    )(h, a, w1, b1, w2, b2, g, bt)


def _final_call(h, a, w1, b1, w2, b2, g, bt, batch, wc, bc):
    return pl.pallas_call(
        _final_body,
        out_shape=jax.ShapeDtypeStruct((G, NCOORD3), jnp.float32),
    )(h, a, w1, b1, w2, b2, g, bt, batch, wc, bc)


def kernel(x, edge_index, batch, emb_table, W1_0, b1_0, W2_0, b2_0, gamma_0,
           beta_0, W1_1, b1_1, W2_1, b2_1, gamma_1, beta_1, W1_2, b1_2, W2_2,
           b2_2, gamma_2, beta_2, Wc, bc):
    zeros = jnp.zeros((DUMP, EMB), jnp.float32)
    batch2 = batch.reshape(N, 1)

    params = [
        (W1_0, b1_0.reshape(1, EMB), W2_0, b2_0.reshape(1, EMB),
         gamma_0.reshape(1, EMB), beta_0.reshape(1, EMB)),
        (W1_1, b1_1.reshape(1, EMB), W2_1, b2_1.reshape(1, EMB),
         gamma_1.reshape(1, EMB), beta_1.reshape(1, EMB)),
        (W1_2, b1_2.reshape(1, EMB), W2_2, b2_2.reshape(1, EMB),
         gamma_2.reshape(1, EMB), beta_2.reshape(1, EMB)),
    ]

    h = _emb_gather(x.reshape(N), emb_table)
    for l, (w1, b1, w2, b2, g, bt) in enumerate(params):
        agg = _edge_agg(edge_index, h, zeros)
        if l < 2:
            h = _mlp_call(h, agg, w1, b1, w2, b2, g, bt, leaky=True)
        else:
            coords = _final_call(h, agg, w1, b1, w2, b2, g, bt,
                                 batch2, Wc, bc.reshape(1, NCOORD3))
    return coords.reshape(-1, 3)


# speedup vs baseline: 1.1983x; 1.0007x over previous
import functools

import jax
import jax.numpy as jnp
from jax import lax
from jax.experimental import pallas as pl
from jax.experimental.pallas import tpu as pltpu
from jax.experimental.pallas import tpu_sc as plsc

N = 10000
E = 320000
EMB = 128
G = 64
NCOORD3 = 273 * 3

NC = 2
NS = 16
NW = NC * NS
CH = 128
NCHUNK = E // CH
DUMP = 640
TAIL = N - (NS - 1) * DUMP
HK = 80
NHCH = N // HK

NBUF = 4
NRB = 3

_mesh = plsc.VectorSubcoreMesh(core_axis_name="c", subcore_axis_name="s")


@functools.partial(
    pl.kernel,
    out_type=jax.ShapeDtypeStruct((N, EMB), jnp.float32),
    mesh=_mesh,
    scratch_types=(
        [pltpu.VMEM((HK,), jnp.int32) for _ in range(2)]
        + [pltpu.VMEM((HK, EMB), jnp.float32) for _ in range(2)]
        + [pltpu.SemaphoreType.DMA for _ in range(2)]
    ),
)
def _emb_gather(x_hbm, emb_hbm, out_hbm, xg0, xg1, r0, r1, sg0, sg1):
    c = lax.axis_index("c")
    s = lax.axis_index("s")
    w = s * NC + c
    xg, rows, sg = [xg0, xg1], [r0, r1], [sg0, sg1]

    for k in range(4):
        ch = w * 4 + k

        @pl.when(ch < NHCH)
        def _():
            base = ch * HK
            pltpu.sync_copy(x_hbm.at[pl.ds(base, HK)], xg[k % 2])
            pltpu.async_copy(emb_hbm.at[xg[k % 2]], rows[k % 2],
                             sg[k % 2]).wait()
            pltpu.sync_copy(rows[k % 2], out_hbm.at[pl.ds(base, HK)])


@functools.partial(
    pl.kernel,
    out_type=jax.ShapeDtypeStruct((NC, N, EMB), jnp.float32),
    mesh=_mesh,
    scratch_types=(
        [pltpu.VMEM((2, CH), jnp.int32) for _ in range(NBUF)]
        + [pltpu.VMEM((CH, EMB), jnp.float32) for _ in range(NRB)]
        + [pltpu.VMEM_SHARED((N, EMB), jnp.float32)]
        + [pltpu.SemaphoreType.DMA for _ in range(NBUF + 2 * NRB)]
    ),
)
def _edge_agg(ei_hbm, h_hbm, zeros_hbm, out_hbm, *sc):
    idxb = sc[0:NBUF]
    rows = sc[NBUF:NBUF + NRB]
    agg_sh = sc[NBUF + NRB]
    p = NBUF + NRB + 1
    sem_i = sc[p:p + NBUF]
    sem_g = sc[p + NBUF:p + NBUF + NRB]
    sem_s = sc[p + NBUF + NRB:p + NBUF + 2 * NRB]

    c = lax.axis_index("c")
    s = lax.axis_index("s")
    w = s * NC + c
    n_i = (NCHUNK - 1 - w) // NW + 1

    def off(i):
        return (w + NW * i) * CH

    def start_idx(i, b):
        pltpu.async_copy(ei_hbm.at[:, pl.ds(off(i), CH)], idxb[b], sem_i[b])

    def wait_idx(i, b):
        pltpu.make_async_copy(ei_hbm.at[:, pl.ds(off(i), CH)], idxb[b],
                              sem_i[b]).wait()

    def gather_desc(b, r):
        return pltpu.make_async_copy(h_hbm.at[idxb[b].at[0]], rows[r],
                                     sem_g[r])

    def scatter_desc(b, r):
        return pltpu.make_async_copy(rows[r], agg_sh.at[idxb[b].at[1]],
                                     sem_s[r])

    start_idx(0, 0)
    start_idx(1, 1)
    start_idx(2, 2)
    wait_idx(0, 0)
    gather_desc(0, 0).start()
    wait_idx(1, 1)
    gather_desc(1, 1).start()

    @pl.when(s < NS - 1)
    def _():
        pltpu.sync_copy(zeros_hbm, agg_sh.at[pl.ds(s * DUMP, DUMP)])

    @pl.when(s == NS - 1)
    def _():
        pltpu.sync_copy(zeros_hbm.at[pl.ds(0, TAIL)],
                        agg_sh.at[pl.ds((NS - 1) * DUMP, TAIL)])

    plsc.subcore_barrier()

    UNROLL = 12
    n_g = (n_i - 1) // UNROLL + 1

    def loop_body(g, carry):
        for j in range(UNROLL):
            i = g * UNROLL + j

            @pl.when(i < n_i)
            def _():
                @pl.when(i >= 1)
                def _():
                    scatter_desc((j - 1) % NBUF, (j - 1) % NRB).wait()

                @pl.when(i + 3 < n_i)
                def _():
                    start_idx(i + 3, (j + 3) % NBUF)

                @pl.when(i + 2 < n_i)
                def _():
                    wait_idx(i + 2, (j + 2) % NBUF)
                    gather_desc((j + 2) % NBUF, (j + 2) % NRB).start()

                gather_desc(j % NBUF, j % NRB).wait()
                pltpu.async_copy(rows[j % NRB], agg_sh.at[idxb[j % NBUF].at[1]],
                                 sem_s[j % NRB], add=True)
        return carry

    lax.fori_loop(0, n_g, loop_body, 0)

    for b in range(NBUF):
        for r in range(NRB):
            @pl.when((lax.rem(n_i - 1, NBUF) == b)
                     & (lax.rem(n_i - 1, NRB) == r))
            def _():
                scatter_desc(b, r).wait()

    plsc.subcore_barrier()

    @pl.when(s < NS - 1)
    def _():
        pltpu.sync_copy(agg_sh.at[pl.ds(s * DUMP, DUMP)],
                        out_hbm.at[c, pl.ds(s * DUMP, DUMP)])

    @pl.when(s == NS - 1)
    def _():
        pltpu.sync_copy(agg_sh.at[pl.ds((NS - 1) * DUMP, TAIL)],
                        out_hbm.at[c, pl.ds((NS - 1) * DUMP, TAIL)])


def _mlp_body(h_ref, a_ref, w1_ref, b1_ref, w2_ref, b2_ref, g_ref,
              bt_ref, o_ref, *, leaky):
    z = h_ref[...] + a_ref[0] + a_ref[1]
    z = jnp.dot(z, w1_ref[...], preferred_element_type=jnp.float32) + b1_ref[...]
    z = jnp.maximum(z, 0.0)
    z = jnp.dot(z, w2_ref[...], preferred_element_type=jnp.float32) + b2_ref[...]
    mu = jnp.mean(z, axis=0, keepdims=True)
    d = z - mu
    var = jnp.mean(d * d, axis=0, keepdims=True)
    zn = g_ref[...] * d * lax.rsqrt(var + 1e-5) + bt_ref[...]
    if leaky:
        zn = jnp.where(zn > 0, zn, 0.01 * zn)
    o_ref[...] = zn


def _final_body(h_ref, a_ref, w1_ref, b1_ref, w2_ref, b2_ref, g_ref,
                bt_ref, batch_ref, wc_ref, bc_ref, o_ref):
    z = h_ref[...] + a_ref[0] + a_ref[1]
    z = jnp.dot(z, w1_ref[...], preferred_element_type=jnp.float32) + b1_ref[...]
    z = jnp.maximum(z, 0.0)
    z = jnp.dot(z, w2_ref[...], preferred_element_type=jnp.float32) + b2_ref[...]
    mu = jnp.mean(z, axis=0, keepdims=True)
    d = z - mu
    var = jnp.mean(d * d, axis=0, keepdims=True)
    zn = g_ref[...] * d * lax.rsqrt(var + 1e-5) + bt_ref[...]

    onehot = jnp.where(
        batch_ref[...] == lax.broadcasted_iota(jnp.int32, (N, G), 1), 1.0, 0.0)
    cnt = jnp.maximum(jnp.sum(onehot, axis=0, keepdims=True), 1.0)
    oh_n = onehot / cnt
    rep = lax.dot_general(oh_n, zn, (((0,), (0,)), ((), ())),
                          preferred_element_type=jnp.float32)
    o_ref[...] = jnp.dot(rep, wc_ref[...],
                         preferred_element_type=jnp.float32) + bc_ref[...]


def _mlp_call(h, a, w1, b1, w2, b2, g, bt, leaky):
    return pl.pallas_call(
        functools.partial(_mlp_body, leaky=leaky),
        out_shape=jax.ShapeDtypeStruct((N, EMB), jnp.float32),
    )(h, a, w1, b1, w2, b2, g, bt)


def _final_call(h, a, w1, b1, w2, b2, g, bt, batch, wc, bc):
    return pl.pallas_call(
        _final_body,
        out_shape=jax.ShapeDtypeStruct((G, NCOORD3), jnp.float32),
    )(h, a, w1, b1, w2, b2, g, bt, batch, wc, bc)


def kernel(x, edge_index, batch, emb_table, W1_0, b1_0, W2_0, b2_0, gamma_0,
           beta_0, W1_1, b1_1, W2_1, b2_1, gamma_1, beta_1, W1_2, b1_2, W2_2,
           b2_2, gamma_2, beta_2, Wc, bc):
    zeros = jnp.zeros((DUMP, EMB), jnp.float32)
    batch2 = batch.reshape(N, 1)

    params = [
        (W1_0, b1_0.reshape(1, EMB), W2_0, b2_0.reshape(1, EMB),
         gamma_0.reshape(1, EMB), beta_0.reshape(1, EMB)),
        (W1_1, b1_1.reshape(1, EMB), W2_1, b2_1.reshape(1, EMB),
         gamma_1.reshape(1, EMB), beta_1.reshape(1, EMB)),
        (W1_2, b1_2.reshape(1, EMB), W2_2, b2_2.reshape(1, EMB),
         gamma_2.reshape(1, EMB), beta_2.reshape(1, EMB)),
    ]

    h = _emb_gather(x.reshape(N), emb_table)
    for l, (w1, b1, w2, b2, g, bt) in enumerate(params):
        agg = _edge_agg(edge_index, h, zeros)
        if l < 2:
            h = _mlp_call(h, agg, w1, b1, w2, b2, g, bt, leaky=True)
        else:
            coords = _final_call(h, agg, w1, b1, w2, b2, g, bt,
                                 batch2, Wc, bc.reshape(1, NCOORD3))
    return coords.reshape(-1, 3)
